# baseline (speedup 1.0000x reference)
import jax
import jax.numpy as jnp
from jax.experimental import pallas as pl

_INPUT_SIZE = 768
_LATENT = 64
_LEVELS = 4
_K = 1024
_B = 16384
_BT = 512

_DEF = jax.lax.Precision.DEFAULT
_HI = jax.lax.Precision.HIGHEST


def _fused_body(x_ref, we0, be0, we1, be1, we2, be2, cb_ref, cn_ref,
                wd0, bd0, wd1, bd1, wd2, bd2,
                dec_ref, r_ref, e_ref, idx_ref, cnt_ref):
    f32 = jnp.float32
    h = jnp.dot(x_ref[...], we0[...], precision=_DEF) + be0[...]
    h = jnp.maximum(h, 0.0)
    h = jnp.dot(h, we1[...], precision=_DEF) + be1[...]
    h = jnp.maximum(h, 0.0)
    z = jnp.dot(h, we2[...], precision=_DEF) + be2[...]

    residual = z
    z_hat = jnp.zeros_like(z)
    idx_cols, cnt_rows = [], []
    iota_k = jax.lax.broadcasted_iota(jnp.int32, (_BT, _K), 1)
    for l in range(_LEVELS):
        cb = cb_ref[l]
        cnorm_row = cn_ref[l]
        scores = jax.lax.dot_general(
            residual, cb, (((1,), (1,)), ((), ())),
            precision=_DEF)
        rnorm = jnp.sum(residual * residual, axis=1, keepdims=True)
        d = (rnorm - 2.0 * scores) + cnorm_row
        dmin = jnp.min(d, axis=1, keepdims=True)
        idx2d = jnp.min(jnp.where(d <= dmin, iota_k, _K), axis=1,
                        keepdims=True)
        onehot = (iota_k == idx2d).astype(f32)
        e_l = jnp.dot(onehot, cb, precision=_HI)
        r_ref[l] = residual
        e_ref[l] = e_l
        idx_cols.append(idx2d)
        cnt_rows.append(jnp.sum(onehot, axis=0, keepdims=True))
        z_hat = z_hat + e_l
        residual = residual - e_l

    idx_ref[...] = jnp.concatenate(idx_cols, axis=1)

    @pl.when(pl.program_id(0) == 0)
    def _init():
        cnt_ref[...] = jnp.zeros_like(cnt_ref)

    cnt_ref[...] += jnp.concatenate(cnt_rows, axis=0)

    h = jnp.dot(z_hat, wd0[...], precision=_DEF) + bd0[...]
    h = jnp.maximum(h, 0.0)
    h = jnp.dot(h, wd1[...], precision=_DEF) + bd1[...]
    h = jnp.maximum(h, 0.0)
    dec_ref[...] = jnp.dot(h, wd2[...], precision=_DEF) + bd2[...]


@jax.jit
def kernel(x, We0, be0, We1, be1, We2, be2, codebooks,
           Wd0, bd0, Wd1, bd1, Wd2, bd2):
    nb = _B // _BT
    cb_norms = jnp.sum(codebooks * codebooks, axis=2)[:, None, :]
    full = lambda shape: pl.BlockSpec(shape, lambda i: (0,) * len(shape))
    out = pl.pallas_call(
        _fused_body,
        grid=(nb,),
        in_specs=[
            pl.BlockSpec((_BT, _INPUT_SIZE), lambda i: (i, 0)),
            full(We0.shape), full((1, 512)),
            full(We1.shape), full((1, 256)),
            full(We2.shape), full((1, _LATENT)),
            full(codebooks.shape),
            full((_LEVELS, 1, _K)),
            full(Wd0.shape), full((1, 256)),
            full(Wd1.shape), full((1, 512)),
            full(Wd2.shape), full((1, _INPUT_SIZE)),
        ],
        out_specs=[
            pl.BlockSpec((_BT, _INPUT_SIZE), lambda i: (i, 0)),
            pl.BlockSpec((_LEVELS, _BT, _LATENT), lambda i: (0, i, 0)),
            pl.BlockSpec((_LEVELS, _BT, _LATENT), lambda i: (0, i, 0)),
            pl.BlockSpec((_BT, _LEVELS), lambda i: (i, 0)),
            pl.BlockSpec((_LEVELS, _K), lambda i: (0, 0)),
        ],
        out_shape=[
            jax.ShapeDtypeStruct((_B, _INPUT_SIZE), jnp.float32),
            jax.ShapeDtypeStruct((_LEVELS, _B, _LATENT), jnp.float32),
            jax.ShapeDtypeStruct((_LEVELS, _B, _LATENT), jnp.float32),
            jax.ShapeDtypeStruct((_B, _LEVELS), jnp.int32),
            jax.ShapeDtypeStruct((_LEVELS, _K), jnp.float32),
        ],
    )(x, We0, be0.reshape(1, -1), We1, be1.reshape(1, -1),
      We2, be2.reshape(1, -1), codebooks, cb_norms,
      Wd0, bd0.reshape(1, -1), Wd1, bd1.reshape(1, -1), Wd2, bd2.reshape(1, -1))
    decoded, r, e, quantized, counts_f = out
    return (decoded, r, e, counts_f.astype(jnp.int32), quantized)


# speedup vs baseline: 1.4372x; 1.4372x over previous
import jax
import jax.numpy as jnp
from jax.experimental import pallas as pl

_INPUT_SIZE = 768
_LATENT = 64
_LEVELS = 4
_K = 1024
_B = 16384
_BT = 512

_DEF = jax.lax.Precision.DEFAULT


def _fused_body(x_ref, we0, be0, we1, be1, we2, be2, cb_ref, cn_ref,
                b0_ref, b1_ref, b2_ref,
                wd0, bd0, wd1, bd1, wd2, bd2,
                dec_ref, r_ref, e_ref, idx_ref, cnt_ref):
    f32 = jnp.float32
    bf16 = jnp.bfloat16
    h = jnp.dot(x_ref[...], we0[...], precision=_DEF) + be0[...]
    h = jnp.maximum(h, 0.0)
    h = jnp.dot(h, we1[...], precision=_DEF) + be1[...]
    h = jnp.maximum(h, 0.0)
    z = jnp.dot(h, we2[...], precision=_DEF) + be2[...]

    residual = z
    z_hat = jnp.zeros_like(z)
    idx_cols, cnt_rows = [], []
    iota_k = jax.lax.broadcasted_iota(jnp.int32, (_BT, _K), 1)
    ones_bt = jnp.ones((1, _BT), dtype=bf16)
    for l in range(_LEVELS):
        cb = cb_ref[l]
        cnorm_row = cn_ref[l]
        scores = jax.lax.dot_general(
            residual, cb, (((1,), (1,)), ((), ())),
            precision=_DEF)
        rnorm = jnp.sum(residual * residual, axis=1, keepdims=True)
        d = (rnorm - 2.0 * scores) + cnorm_row
        dmin = jnp.min(d, axis=1, keepdims=True)
        idx2d = jnp.min(jnp.where(d <= dmin, iota_k, _K), axis=1,
                        keepdims=True)
        onehot = (iota_k == idx2d).astype(bf16)
        e_l = (jnp.dot(onehot, b0_ref[l], preferred_element_type=f32)
               + jnp.dot(onehot, b1_ref[l], preferred_element_type=f32)
               ) + jnp.dot(onehot, b2_ref[l], preferred_element_type=f32)
        r_ref[l] = residual
        e_ref[l] = e_l
        idx_cols.append(idx2d)
        cnt_rows.append(jnp.dot(ones_bt, onehot,
                                preferred_element_type=f32))
        z_hat = z_hat + e_l
        residual = residual - e_l

    idx_ref[...] = jnp.concatenate(idx_cols, axis=1)

    @pl.when(pl.program_id(0) == 0)
    def _init():
        cnt_ref[...] = jnp.zeros_like(cnt_ref)

    cnt_ref[...] += jnp.concatenate(cnt_rows, axis=0)

    h = jnp.dot(z_hat, wd0[...], precision=_DEF) + bd0[...]
    h = jnp.maximum(h, 0.0)
    h = jnp.dot(h, wd1[...], precision=_DEF) + bd1[...]
    h = jnp.maximum(h, 0.0)
    dec_ref[...] = jnp.dot(h, wd2[...], precision=_DEF) + bd2[...]


@jax.jit
def kernel(x, We0, be0, We1, be1, We2, be2, codebooks,
           Wd0, bd0, Wd1, bd1, Wd2, bd2):
    nb = _B // _BT
    f32 = jnp.float32
    bf16 = jnp.bfloat16
    cb_norms = jnp.sum(codebooks * codebooks, axis=2)[:, None, :]
    b0 = codebooks.astype(bf16)
    r1 = codebooks - b0.astype(f32)
    b1 = r1.astype(bf16)
    b2 = (r1 - b1.astype(f32)).astype(bf16)
    full = lambda shape: pl.BlockSpec(shape, lambda i: (0,) * len(shape))
    cbs = (_LEVELS, _K, _LATENT)
    out = pl.pallas_call(
        _fused_body,
        grid=(nb,),
        in_specs=[
            pl.BlockSpec((_BT, _INPUT_SIZE), lambda i: (i, 0)),
            full(We0.shape), full((1, 512)),
            full(We1.shape), full((1, 256)),
            full(We2.shape), full((1, _LATENT)),
            full(cbs),
            full((_LEVELS, 1, _K)),
            full(cbs), full(cbs), full(cbs),
            full(Wd0.shape), full((1, 256)),
            full(Wd1.shape), full((1, 512)),
            full(Wd2.shape), full((1, _INPUT_SIZE)),
        ],
        out_specs=[
            pl.BlockSpec((_BT, _INPUT_SIZE), lambda i: (i, 0)),
            pl.BlockSpec((_LEVELS, _BT, _LATENT), lambda i: (0, i, 0)),
            pl.BlockSpec((_LEVELS, _BT, _LATENT), lambda i: (0, i, 0)),
            pl.BlockSpec((_BT, _LEVELS), lambda i: (i, 0)),
            pl.BlockSpec((_LEVELS, _K), lambda i: (0, 0)),
        ],
        out_shape=[
            jax.ShapeDtypeStruct((_B, _INPUT_SIZE), jnp.float32),
            jax.ShapeDtypeStruct((_LEVELS, _B, _LATENT), jnp.float32),
            jax.ShapeDtypeStruct((_LEVELS, _B, _LATENT), jnp.float32),
            jax.ShapeDtypeStruct((_B, _LEVELS), jnp.int32),
            jax.ShapeDtypeStruct((_LEVELS, _K), jnp.float32),
        ],
    )(x, We0, be0.reshape(1, -1), We1, be1.reshape(1, -1),
      We2, be2.reshape(1, -1), codebooks, cb_norms, b0, b1, b2,
      Wd0, bd0.reshape(1, -1), Wd1, bd1.reshape(1, -1), Wd2, bd2.reshape(1, -1))
    decoded, r, e, quantized, counts_f = out
    return (decoded, r, e, counts_f.astype(jnp.int32), quantized)


# speedup vs baseline: 1.6531x; 1.1502x over previous
import jax
import jax.numpy as jnp
from jax.experimental import pallas as pl

_INPUT_SIZE = 768
_LATENT = 64
_LEVELS = 4
_K = 1024
_B = 16384
_BT = 512
_H = _BT // 2

_DEF = jax.lax.Precision.DEFAULT


def _fused_body(x_ref, we0, be0, we1, be1, we2, be2, cn_ref,
                b0_ref, b1_ref, b2_ref,
                wd0, bd0, wd1, bd1, wd2, bd2,
                dec_ref, r_ref, e_ref, idx_ref, cnt_ref):
    f32 = jnp.float32
    bf16 = jnp.bfloat16
    h = jnp.dot(x_ref[...], we0[...], precision=_DEF) + be0[...]
    h = jnp.maximum(h, 0.0)
    h = jnp.dot(h, we1[...], precision=_DEF) + be1[...]
    h = jnp.maximum(h, 0.0)
    z = jnp.dot(h, we2[...], precision=_DEF) + be2[...]

    residual = [z[:_H], z[_H:]]
    z_hat = [jnp.zeros_like(residual[0]), jnp.zeros_like(residual[1])]
    idx_cols = [[], []]
    cnt_rows = []
    iota_k = jax.lax.broadcasted_iota(jnp.int32, (_H, _K), 1)
    ones_h = jnp.ones((1, _H), dtype=bf16)
    for l in range(_LEVELS):
        b0 = b0_ref[l]
        cnorm_row = cn_ref[l]
        cnt_l = None
        for s in range(2):
            res = residual[s]
            scores = jax.lax.dot_general(
                res.astype(bf16), b0, (((1,), (1,)), ((), ())),
                precision=_DEF, preferred_element_type=f32)
            rnorm = jnp.sum(res * res, axis=1, keepdims=True)
            d = (rnorm - 2.0 * scores) + cnorm_row
            dmin = jnp.min(d, axis=1, keepdims=True)
            idx2d = jnp.min(jnp.where(d <= dmin, iota_k, _K), axis=1,
                            keepdims=True)
            onehot = (iota_k == idx2d).astype(bf16)
            e_l = (jnp.dot(onehot, b0, preferred_element_type=f32)
                   + jnp.dot(onehot, b1_ref[l], preferred_element_type=f32)
                   ) + jnp.dot(onehot, b2_ref[l], preferred_element_type=f32)
            r_ref[l, s * _H:(s + 1) * _H] = res
            e_ref[l, s * _H:(s + 1) * _H] = e_l
            idx_cols[s].append(idx2d)
            cnt_s = jnp.dot(ones_h, onehot, preferred_element_type=f32)
            cnt_l = cnt_s if cnt_l is None else cnt_l + cnt_s
            z_hat[s] = z_hat[s] + e_l
            residual[s] = res - e_l
        cnt_rows.append(cnt_l)

    idx_ref[:_H] = jnp.concatenate(idx_cols[0], axis=1)
    idx_ref[_H:] = jnp.concatenate(idx_cols[1], axis=1)

    @pl.when(pl.program_id(0) == 0)
    def _init():
        cnt_ref[...] = jnp.zeros_like(cnt_ref)

    cnt_ref[...] += jnp.concatenate(cnt_rows, axis=0)

    zh = jnp.concatenate(z_hat, axis=0)
    h = jnp.dot(zh, wd0[...], precision=_DEF) + bd0[...]
    h = jnp.maximum(h, 0.0)
    h = jnp.dot(h, wd1[...], precision=_DEF) + bd1[...]
    h = jnp.maximum(h, 0.0)
    dec_ref[...] = jnp.dot(h, wd2[...], precision=_DEF) + bd2[...]


@jax.jit
def kernel(x, We0, be0, We1, be1, We2, be2, codebooks,
           Wd0, bd0, Wd1, bd1, Wd2, bd2):
    nb = _B // _BT
    f32 = jnp.float32
    bf16 = jnp.bfloat16
    cb_norms = jnp.sum(codebooks * codebooks, axis=2)[:, None, :]
    b0 = codebooks.astype(bf16)
    r1 = codebooks - b0.astype(f32)
    b1 = r1.astype(bf16)
    b2 = (r1 - b1.astype(f32)).astype(bf16)
    full = lambda shape: pl.BlockSpec(shape, lambda i: (0,) * len(shape))
    cbs = (_LEVELS, _K, _LATENT)
    out = pl.pallas_call(
        _fused_body,
        grid=(nb,),
        in_specs=[
            pl.BlockSpec((_BT, _INPUT_SIZE), lambda i: (i, 0)),
            full(We0.shape), full((1, 512)),
            full(We1.shape), full((1, 256)),
            full(We2.shape), full((1, _LATENT)),
            full((_LEVELS, 1, _K)),
            full(cbs), full(cbs), full(cbs),
            full(Wd0.shape), full((1, 256)),
            full(Wd1.shape), full((1, 512)),
            full(Wd2.shape), full((1, _INPUT_SIZE)),
        ],
        out_specs=[
            pl.BlockSpec((_BT, _INPUT_SIZE), lambda i: (i, 0)),
            pl.BlockSpec((_LEVELS, _BT, _LATENT), lambda i: (0, i, 0)),
            pl.BlockSpec((_LEVELS, _BT, _LATENT), lambda i: (0, i, 0)),
            pl.BlockSpec((_BT, _LEVELS), lambda i: (i, 0)),
            pl.BlockSpec((_LEVELS, _K), lambda i: (0, 0)),
        ],
        out_shape=[
            jax.ShapeDtypeStruct((_B, _INPUT_SIZE), jnp.float32),
            jax.ShapeDtypeStruct((_LEVELS, _B, _LATENT), jnp.float32),
            jax.ShapeDtypeStruct((_LEVELS, _B, _LATENT), jnp.float32),
            jax.ShapeDtypeStruct((_B, _LEVELS), jnp.int32),
            jax.ShapeDtypeStruct((_LEVELS, _K), jnp.float32),
        ],
    )(x, We0, be0.reshape(1, -1), We1, be1.reshape(1, -1),
      We2, be2.reshape(1, -1), cb_norms, b0, b1, b2,
      Wd0, bd0.reshape(1, -1), Wd1, bd1.reshape(1, -1), Wd2, bd2.reshape(1, -1))
    decoded, r, e, quantized, counts_f = out
    return (decoded, r, e, counts_f.astype(jnp.int32), quantized)


# speedup vs baseline: 2.1821x; 1.3200x over previous
import jax
import jax.numpy as jnp
from jax.experimental import pallas as pl

_INPUT_SIZE = 768
_LATENT = 64
_LEVELS = 4
_K = 1024
_B = 16384
_BT = 512
_H = _BT // 2

_DEF = jax.lax.Precision.DEFAULT


def _fused_body(x_ref, we0, be0, we1, be1, we2, be2, cn_ref,
                b0_ref, bcat_ref,
                wd0, bd0, wd1, bd1, wd2, bd2,
                dec_ref, r_ref, e_ref, idx_ref, cnt_ref):
    f32 = jnp.float32
    bf16 = jnp.bfloat16
    h = jnp.dot(x_ref[...], we0[...], precision=_DEF) + be0[...]
    h = jnp.maximum(h, 0.0)
    h = jnp.dot(h, we1[...], precision=_DEF) + be1[...]
    h = jnp.maximum(h, 0.0)
    z = jnp.dot(h, we2[...], precision=_DEF) + be2[...]

    residual = [z[:_H], z[_H:]]
    z_hat = [jnp.zeros_like(residual[0]), jnp.zeros_like(residual[1])]
    idx_cols = [[], []]
    cnt_rows = []
    iota_k = jax.lax.broadcasted_iota(jnp.int32, (_H, _K), 1)
    ones_h = jnp.ones((1, _H), dtype=bf16)
    for l in range(_LEVELS):
        b0 = b0_ref[l]
        cnorm_row = cn_ref[l]
        cnt_l = None
        for s in range(2):
            res = residual[s]
            scores = jax.lax.dot_general(
                res.astype(bf16), b0, (((1,), (1,)), ((), ())),
                precision=_DEF, preferred_element_type=f32)
            rnorm = jnp.sum(res * res, axis=1, keepdims=True)
            d = (rnorm - 2.0 * scores) + cnorm_row
            dmin = jnp.min(d, axis=1, keepdims=True)
            idx2d = jnp.min(jnp.where(d <= dmin, iota_k, _K), axis=1,
                            keepdims=True)
            onehot = (iota_k == idx2d).astype(bf16)
            e3 = jnp.dot(onehot, bcat_ref[l],
                         preferred_element_type=f32)
            e_l = ((e3[:, :_LATENT] + e3[:, _LATENT:2 * _LATENT])
                   + e3[:, 2 * _LATENT:])
            r_ref[l, s * _H:(s + 1) * _H] = res
            e_ref[l, s * _H:(s + 1) * _H] = e_l
            idx_cols[s].append(idx2d)
            cnt_s = jnp.dot(ones_h, onehot, preferred_element_type=f32)
            cnt_l = cnt_s if cnt_l is None else cnt_l + cnt_s
            z_hat[s] = z_hat[s] + e_l
            residual[s] = res - e_l
        cnt_rows.append(cnt_l)

    idx_ref[:_H] = jnp.concatenate(idx_cols[0], axis=1)
    idx_ref[_H:] = jnp.concatenate(idx_cols[1], axis=1)

    @pl.when(pl.program_id(0) == 0)
    def _init():
        cnt_ref[...] = jnp.zeros_like(cnt_ref)

    cnt_ref[...] += jnp.concatenate(cnt_rows, axis=0)

    zh = jnp.concatenate(z_hat, axis=0)
    h = jnp.dot(zh, wd0[...], precision=_DEF) + bd0[...]
    h = jnp.maximum(h, 0.0)
    h = jnp.dot(h, wd1[...], precision=_DEF) + bd1[...]
    h = jnp.maximum(h, 0.0)
    dec_ref[...] = jnp.dot(h, wd2[...], precision=_DEF) + bd2[...]


@jax.jit
def kernel(x, We0, be0, We1, be1, We2, be2, codebooks,
           Wd0, bd0, Wd1, bd1, Wd2, bd2):
    nb = _B // _BT
    f32 = jnp.float32
    bf16 = jnp.bfloat16
    cb_norms = jnp.sum(codebooks * codebooks, axis=2)[:, None, :]
    b0 = codebooks.astype(bf16)
    r1 = codebooks - b0.astype(f32)
    b1 = r1.astype(bf16)
    b2 = (r1 - b1.astype(f32)).astype(bf16)
    bcat = jnp.concatenate([b0, b1, b2], axis=2)
    full = lambda shape: pl.BlockSpec(shape, lambda i: (0,) * len(shape))
    cbs = (_LEVELS, _K, _LATENT)
    out = pl.pallas_call(
        _fused_body,
        grid=(nb,),
        in_specs=[
            pl.BlockSpec((_BT, _INPUT_SIZE), lambda i: (i, 0)),
            full(We0.shape), full((1, 512)),
            full(We1.shape), full((1, 256)),
            full(We2.shape), full((1, _LATENT)),
            full((_LEVELS, 1, _K)),
            full(cbs), full((_LEVELS, _K, 3 * _LATENT)),
            full(Wd0.shape), full((1, 256)),
            full(Wd1.shape), full((1, 512)),
            full(Wd2.shape), full((1, _INPUT_SIZE)),
        ],
        out_specs=[
            pl.BlockSpec((_BT, _INPUT_SIZE), lambda i: (i, 0)),
            pl.BlockSpec((_LEVELS, _BT, _LATENT), lambda i: (0, i, 0)),
            pl.BlockSpec((_LEVELS, _BT, _LATENT), lambda i: (0, i, 0)),
            pl.BlockSpec((_BT, _LEVELS), lambda i: (i, 0)),
            pl.BlockSpec((_LEVELS, _K), lambda i: (0, 0)),
        ],
        out_shape=[
            jax.ShapeDtypeStruct((_B, _INPUT_SIZE), jnp.float32),
            jax.ShapeDtypeStruct((_LEVELS, _B, _LATENT), jnp.float32),
            jax.ShapeDtypeStruct((_LEVELS, _B, _LATENT), jnp.float32),
            jax.ShapeDtypeStruct((_B, _LEVELS), jnp.int32),
            jax.ShapeDtypeStruct((_LEVELS, _K), jnp.float32),
        ],
    )(x, We0, be0.reshape(1, -1), We1, be1.reshape(1, -1),
      We2, be2.reshape(1, -1), cb_norms, b0, bcat,
      Wd0, bd0.reshape(1, -1), Wd1, bd1.reshape(1, -1), Wd2, bd2.reshape(1, -1))
    decoded, r, e, quantized, counts_f = out
    return (decoded, r, e, counts_f.astype(jnp.int32), quantized)
